# Initial kernel scaffold; baseline (speedup 1.0000x reference)
import jax
import jax.numpy as jnp
from jax.experimental import pallas as pl


def kernel(inputs, embed_tables, W0, b0, W1, b1, W2, b2, W3, b3):
    raise NotImplementedError("write your pallas kernel here")


# speedup vs baseline: 1.2107x; 1.2107x over previous
import functools

import jax
import jax.numpy as jnp
from jax import lax
from jax.experimental import pallas as pl
from jax.experimental.pallas import tpu as pltpu
from jax.experimental.pallas import tpu_sc as plsc

B = 4096
ND = 13
NS = 26
VOCAB = 100000
ED = 32

NW = 32
CHUNK = 128
TOT_ROWS = B * NS
CPW = TOT_ROWS // (NW * CHUNK)


def _sc_gather_body(table_hbm, idx_hbm, out_hbm, idx_v, rows_v, sem):
    wid = lax.axis_index("s") * 2 + lax.axis_index("c")
    pltpu.sync_copy(idx_hbm.at[wid], idx_v)
    copies = [
        pltpu.async_copy(table_hbm.at[idx_v.at[j]], rows_v.at[j], sem)
        for j in range(CPW)
    ]
    for c in copies:
        c.wait()
    pltpu.sync_copy(rows_v, out_hbm.at[wid])


def _sc_gather(table_flat, idx3d):
    mesh = plsc.VectorSubcoreMesh(core_axis_name="c", subcore_axis_name="s")
    k = functools.partial(
        pl.kernel,
        mesh=mesh,
        out_type=jax.ShapeDtypeStruct((NW, CPW, CHUNK, ED), jnp.float32),
        scratch_types=[
            pltpu.VMEM((CPW, CHUNK), jnp.int32),
            pltpu.VMEM((CPW, CHUNK, ED), jnp.float32),
            pltpu.SemaphoreType.DMA,
        ],
        compiler_params=pltpu.CompilerParams(use_tc_tiling_on_sc=False),
    )(_sc_gather_body)
    return k(table_flat, idx3d)


BM = 512


def _mlp_body(dense_ref, embed_ref, w0a, w0b, b0, w1, b1, w2, b2, w3, b3,
              out_ref):
    f32 = jnp.float32
    x0 = jnp.dot(dense_ref[...], w0a[...], preferred_element_type=f32)
    x0 += jnp.dot(embed_ref[...], w0b[...], preferred_element_type=f32)
    h = jnp.maximum(x0 + b0[...], 0.0)
    h = jnp.maximum(
        jnp.dot(h, w1[...], preferred_element_type=f32) + b1[...], 0.0)
    h = jnp.maximum(
        jnp.dot(h, w2[...], preferred_element_type=f32) + b2[...], 0.0)
    out_ref[...] = jnp.dot(h, w3[...], preferred_element_type=f32) + b3[...]


def _mlp(dense, embed, w0a, w0b, b0, w1, b1, w2, b2, w3, b3):
    nb = B // BM
    full = lambda shape: pl.BlockSpec(shape, lambda i: (0, 0))
    return pl.pallas_call(
        _mlp_body,
        grid=(nb,),
        in_specs=[
            pl.BlockSpec((BM, ND), lambda i: (i, 0)),
            pl.BlockSpec((BM, NS * ED), lambda i: (i, 0)),
            full(w0a.shape),
            full(w0b.shape),
            full(b0.shape),
            full(w1.shape),
            full(b1.shape),
            full(w2.shape),
            full(b2.shape),
            full(w3.shape),
            full(b3.shape),
        ],
        out_specs=pl.BlockSpec((BM, 256), lambda i: (i, 0)),
        out_shape=jax.ShapeDtypeStruct((B, 256), jnp.float32),
    )(dense, embed, w0a, w0b, b0, w1, b1, w2, b2, w3, b3)


def kernel(inputs, embed_tables, W0, b0, W1, b1, W2, b2, W3, b3):
    dense = inputs[:, :ND]
    idx = inputs[:, ND:].astype(jnp.int32)
    flat_idx = (idx + jnp.arange(NS, dtype=jnp.int32) * VOCAB).reshape(-1)
    idx3d = flat_idx.reshape(NW, CPW, CHUNK)
    table_flat = embed_tables.reshape(NS * VOCAB, ED)

    rows = _sc_gather(table_flat, idx3d)
    embed = rows.reshape(B, NS * ED)

    w0a = W0[:ND]
    w0b = W0[ND:]
    out = _mlp(dense, embed, w0a, w0b, b0.reshape(1, -1), W1,
               b1.reshape(1, -1), W2, b2.reshape(1, -1), W3,
               b3.reshape(1, -1))
    return out
